# baseline (speedup 1.0000x reference)
import jax
import jax.numpy as jnp
from jax.experimental import pallas as pl
from jax.experimental.pallas import tpu as pltpu

_SUBROWS = 8
_NSTREAM = 4


def _mrr_body(*refs):
    x_refs = refs[:_NSTREAM]
    tgt_ref, pm_ref, out_ref, acc_ref = refs[_NSTREAM:]
    i = pl.program_id(0)

    @pl.when(i == 0)
    def _init():
        acc_ref[0] = 0.0
        acc_ref[1] = 0.0

    num = jnp.float32(0.0)
    den = jnp.float32(0.0)
    for k in range(_NSTREAM):
        x = x_refs[k][...]
        tgt = tgt_ref[k * _SUBROWS:(k + 1) * _SUBROWS, :]
        pm = pm_ref[k * _SUBROWS:(k + 1) * _SUBROWS, :].astype(jnp.float32)

        iota = jax.lax.broadcasted_iota(jnp.int32, x.shape, 1)
        is_t = iota == tgt
        t = jnp.max(jnp.where(is_t, x, -jnp.inf), axis=1, keepdims=True)
        beats = (x > t) | ((x == t) & (iota < tgt))
        cnt = jnp.sum(beats.astype(jnp.float32), axis=1, keepdims=True)
        num += jnp.sum(pm / (cnt + 1.0))
        den += jnp.sum(pm)

    acc_ref[0] += num
    acc_ref[1] += den

    @pl.when(i == pl.num_programs(0) - 1)
    def _fin():
        out_ref[...] = jnp.full((1, 1), acc_ref[0] / acc_ref[1], jnp.float32)


def kernel(logits, targets, padding_mask):
    B, V = logits.shape
    rows_per_step = _SUBROWS * _NSTREAM
    tgt2d = targets.astype(jnp.int32).reshape(B, 1)
    pm2d = padding_mask.astype(jnp.int32).reshape(B, 1)
    grid = B // rows_per_step

    def x_spec(k):
        return pl.BlockSpec((_SUBROWS, V), lambda i, k=k: (_NSTREAM * i + k, 0))

    out = pl.pallas_call(
        _mrr_body,
        grid=(grid,),
        in_specs=[x_spec(k) for k in range(_NSTREAM)] + [
            pl.BlockSpec((rows_per_step, 1), lambda i: (i, 0)),
            pl.BlockSpec((rows_per_step, 1), lambda i: (i, 0)),
        ],
        out_specs=pl.BlockSpec((1, 1), lambda i: (0, 0)),
        out_shape=jax.ShapeDtypeStruct((1, 1), jnp.float32),
        scratch_shapes=[pltpu.SMEM((2,), jnp.float32)],
        compiler_params=pltpu.CompilerParams(
            dimension_semantics=("arbitrary",),
        ),
    )(*([logits] * _NSTREAM), tgt2d, pm2d)
    return out.reshape(())


# speedup vs baseline: 1.0168x; 1.0168x over previous
import jax
import jax.numpy as jnp
from jax.experimental import pallas as pl
from jax.experimental.pallas import tpu as pltpu

_ROWS = 64


def _mrr_body(x_ref, tgt_ref, pm_ref, out_ref, acc_ref):
    i = pl.program_id(0)

    @pl.when(i == 0)
    def _init():
        acc_ref[0] = 0.0
        acc_ref[1] = 0.0

    x = x_ref[...]
    tgt = tgt_ref[...]
    pm = pm_ref[...].astype(jnp.float32)

    iota = jax.lax.broadcasted_iota(jnp.int32, x.shape, 1)
    t = jnp.max(jnp.where(iota == tgt, x, -jnp.inf), axis=1, keepdims=True)
    beats = (x > t) | ((x == t) & (iota < tgt))
    cnt = jnp.sum(beats.astype(jnp.float32), axis=1, keepdims=True)
    rr = pm / (cnt + 1.0)

    acc_ref[0] += jnp.sum(rr)
    acc_ref[1] += jnp.sum(pm)

    @pl.when(i == pl.num_programs(0) - 1)
    def _fin():
        out_ref[...] = jnp.full((1, 1), acc_ref[0] / acc_ref[1], jnp.float32)


def kernel(logits, targets, padding_mask):
    B, V = logits.shape
    tgt2d = targets.astype(jnp.int32).reshape(B, 1)
    pm2d = padding_mask.astype(jnp.int32).reshape(B, 1)
    grid = B // _ROWS
    out = pl.pallas_call(
        _mrr_body,
        grid=(grid,),
        in_specs=[
            pl.BlockSpec((_ROWS, V), lambda i: (i, 0)),
            pl.BlockSpec((_ROWS, 1), lambda i: (i, 0)),
            pl.BlockSpec((_ROWS, 1), lambda i: (i, 0)),
        ],
        out_specs=pl.BlockSpec((1, 1), lambda i: (0, 0)),
        out_shape=jax.ShapeDtypeStruct((1, 1), jnp.float32),
        scratch_shapes=[pltpu.SMEM((2,), jnp.float32)],
        compiler_params=pltpu.CompilerParams(
            dimension_semantics=("arbitrary",),
        ),
    )(logits, tgt2d, pm2d)
    return out.reshape(())
